# baseline (speedup 1.0000x reference)
import functools

import jax
import jax.numpy as jnp
from jax import lax
from jax.experimental import pallas as pl
from jax.experimental.pallas import tpu as pltpu
from jax.experimental.pallas import tpu_sc as plsc

NUM_FIELDS = 26
VOCAB = 100000
EMBED_DIM = 32
BATCH = 16384

L = 16
NC, NS = 2, 16
NW = NC * NS
ROWS = BATCH * NUM_FIELDS
RPW = ROWS // NW
REPS = 16
CHUNK = NUM_FIELDS * REPS
NCHUNK = RPW // CHUNK
HALVES = EMBED_DIM // L


@functools.partial(
    pl.kernel,
    mesh=plsc.VectorSubcoreMesh(core_axis_name="c", subcore_axis_name="s"),
    out_type=jax.ShapeDtypeStruct((ROWS, EMBED_DIM), jnp.float32),
    compiler_params=pltpu.CompilerParams(use_tc_tiling_on_sc=False),
    scratch_types=[
        pltpu.VMEM((CHUNK,), jnp.int32),
        pltpu.VMEM((CHUNK,), jnp.int32),
        pltpu.VMEM((CHUNK,), jnp.int32),
        pltpu.VMEM((NUM_FIELDS, EMBED_DIM), jnp.float32),
        pltpu.VMEM((CHUNK, EMBED_DIM), jnp.float32),
        pltpu.SemaphoreType.DMA,
    ],
)
def _embed_sc(idx_hbm, table_hbm, col_hbm, offs_hbm, out_hbm,
              raw_v, gidx_v, off_v, bias_v, rows_v, sem):
    wid = lax.axis_index("s") * NC + lax.axis_index("c")
    base = wid * RPW
    pltpu.sync_copy(offs_hbm, off_v)
    pltpu.sync_copy(col_hbm, bias_v)

    def chunk_body(g, carry):
        r0 = base + g * CHUNK
        pltpu.sync_copy(idx_hbm.at[pl.ds(r0, CHUNK)], raw_v)
        for j in range(CHUNK // L):
            s = pl.ds(j * L, L)
            gidx_v[s] = raw_v[s] + off_v[s]
        pltpu.async_copy(table_hbm.at[gidx_v], rows_v, sem).wait()

        def rep_body(rep, c2):
            row0 = rep * NUM_FIELDS
            for f in range(NUM_FIELDS):
                for h in range(HALVES):
                    s = pl.ds(h * L, L)
                    rows_v[row0 + f, s] = rows_v[row0 + f, s] + bias_v[f, s]
            return c2

        lax.fori_loop(0, REPS, rep_body, 0)
        pltpu.sync_copy(rows_v, out_hbm.at[pl.ds(r0, CHUNK)])
        return carry

    lax.fori_loop(0, NCHUNK, chunk_body, 0)


def kernel(categorical_inputs, tables, column_embedding):
    idx_flat = categorical_inputs.astype(jnp.int32).reshape(ROWS)
    table_flat = tables.reshape(NUM_FIELDS * VOCAB, EMBED_DIM)
    offs = jnp.tile(
        jnp.arange(NUM_FIELDS, dtype=jnp.int32) * VOCAB, REPS)
    out = _embed_sc(idx_flat, table_flat, column_embedding, offs)
    return out.reshape(BATCH, NUM_FIELDS, EMBED_DIM)


# speedup vs baseline: 3.4766x; 3.4766x over previous
import functools

import jax
import jax.numpy as jnp
from jax import lax
from jax.experimental import pallas as pl
from jax.experimental.pallas import tpu as pltpu
from jax.experimental.pallas import tpu_sc as plsc

NUM_FIELDS = 26
VOCAB = 100000
EMBED_DIM = 32
BATCH = 16384

L = 16
NC, NS = 2, 16
NW = NC * NS
PLANES = NUM_FIELDS * EMBED_DIM
PPW = PLANES // NW
OCHUNK = 4096
NOC = BATCH // OCHUNK


@functools.partial(
    pl.kernel,
    mesh=plsc.VectorSubcoreMesh(core_axis_name="c", subcore_axis_name="s"),
    out_type=jax.ShapeDtypeStruct((NUM_FIELDS, EMBED_DIM, BATCH),
                                  jnp.float32),
    compiler_params=pltpu.CompilerParams(needs_layout_passes=False),
    scratch_types=[
        pltpu.VMEM((BATCH,), jnp.int32),
        pltpu.VMEM((VOCAB,), jnp.float32),
        pltpu.VMEM((PPW * L,), jnp.float32),
        pltpu.VMEM((OCHUNK,), jnp.float32),
    ],
)
def _embed_sc(idx_hbm, table_hbm, bias_hbm, out_hbm,
              idx_v, plane_v, bias_v, out_v):
    wid = lax.axis_index("s") * NC + lax.axis_index("c")
    p0 = wid * PPW
    pltpu.sync_copy(bias_hbm.at[pl.ds(p0 * L, PPW * L)], bias_v)

    def plane_body(k, f_prev):
        p = p0 + k
        f = p // EMBED_DIM
        d = lax.rem(p, EMBED_DIM)

        pltpu.sync_copy(idx_hbm.at[f, :], idx_v)
        pltpu.sync_copy(table_hbm.at[f, d, :], plane_v)
        bvec = bias_v[pl.ds(k * L, L)]

        for c in range(NOC):
            base = c * OCHUNK

            @pl.loop(0, OCHUNK // L, unroll=8)
            def _(i):
                iv = idx_v[pl.ds(base + i * L, L)]
                vals = plsc.load_gather(plane_v, [iv])
                out_v[pl.ds(i * L, L)] = vals + bvec

            pltpu.sync_copy(out_v, out_hbm.at[f, d, pl.ds(base, OCHUNK)])
        return f

    lax.fori_loop(0, PPW, plane_body, jnp.int32(-1))


def kernel(categorical_inputs, tables, column_embedding):
    table_t = jnp.transpose(tables, (0, 2, 1))
    idx_t = jnp.transpose(categorical_inputs.astype(jnp.int32),
                          (1, 0))
    bias_b = jnp.broadcast_to(
        column_embedding.reshape(PLANES, 1), (PLANES, L)).reshape(-1)
    out_t = _embed_sc(idx_t, table_t, bias_b)
    return jnp.transpose(out_t, (2, 0, 1))


# speedup vs baseline: 3.8920x; 1.1195x over previous
import functools

import jax
import jax.numpy as jnp
from jax import lax
from jax.experimental import pallas as pl
from jax.experimental.pallas import tpu as pltpu
from jax.experimental.pallas import tpu_sc as plsc

NUM_FIELDS = 26
VOCAB = 100000
EMBED_DIM = 32
BATCH = 16384

L = 16
NC, NS = 2, 16
NW = NC * NS
PLANES = NUM_FIELDS * EMBED_DIM
PPW = PLANES // NW
OCHUNK = 4096
NOC = BATCH // OCHUNK


@functools.partial(
    pl.kernel,
    mesh=plsc.VectorSubcoreMesh(core_axis_name="c", subcore_axis_name="s"),
    out_type=jax.ShapeDtypeStruct((NUM_FIELDS, EMBED_DIM, BATCH),
                                  jnp.float32),
    compiler_params=pltpu.CompilerParams(needs_layout_passes=False),
    scratch_types=[
        pltpu.VMEM((BATCH,), jnp.int32),
        pltpu.VMEM((VOCAB,), jnp.float32),
        pltpu.VMEM((PPW * L,), jnp.float32),
        pltpu.VMEM((OCHUNK,), jnp.float32),
        pltpu.VMEM((OCHUNK,), jnp.float32),
        pltpu.SemaphoreType.DMA,
        pltpu.SemaphoreType.DMA,
    ],
)
def _embed_sc(idx_hbm, table_hbm, bias_hbm, out_hbm,
              idx_v, plane_v, bias_v, out_a, out_b, sem_a, sem_b):
    wid = lax.axis_index("s") * NC + lax.axis_index("c")
    p0 = wid * PPW
    pltpu.sync_copy(bias_hbm.at[pl.ds(p0 * L, PPW * L)], bias_v)

    def plane_body(k, f_prev):
        p = p0 + k
        f = p // EMBED_DIM
        d = lax.rem(p, EMBED_DIM)

        @pl.when(f != f_prev)
        def _():
            pltpu.sync_copy(idx_hbm.at[f, :], idx_v)

        pltpu.sync_copy(table_hbm.at[f, d, :], plane_v)
        bvec = bias_v[pl.ds(k * L, L)]

        stores = []
        for c in range(NOC):
            base = c * OCHUNK
            out_v = out_a if c % 2 == 0 else out_b
            sem = sem_a if c % 2 == 0 else sem_b
            if c >= 2:
                stores[c - 2].wait()

            @pl.loop(0, OCHUNK // L, unroll=16)
            def _(i):
                iv = idx_v[pl.ds(base + i * L, L)]
                vals = plsc.load_gather(plane_v, [iv])
                out_v[pl.ds(i * L, L)] = vals + bvec

            stores.append(pltpu.async_copy(
                out_v, out_hbm.at[f, d, pl.ds(base, OCHUNK)], sem))
        stores[-2].wait()
        stores[-1].wait()
        return f

    lax.fori_loop(0, PPW, plane_body, jnp.int32(-1))


def kernel(categorical_inputs, tables, column_embedding):
    table_t = jnp.transpose(tables, (0, 2, 1))
    idx_t = jnp.transpose(categorical_inputs.astype(jnp.int32),
                          (1, 0))
    bias_b = jnp.broadcast_to(
        column_embedding.reshape(PLANES, 1), (PLANES, L)).reshape(-1)
    out_t = _embed_sc(idx_t, table_t, bias_b)
    return jnp.transpose(out_t, (2, 0, 1))


# speedup vs baseline: 8.4511x; 2.1714x over previous
import functools

import jax
import jax.numpy as jnp
from jax import lax
from jax.experimental import pallas as pl
from jax.experimental.pallas import tpu as pltpu
from jax.experimental.pallas import tpu_sc as plsc

NUM_FIELDS = 26
VOCAB = 100000
EMBED_DIM = 32
BATCH = 16384

L = 16
NC, NS = 2, 16
NW = NC * NS
PLANES = NUM_FIELDS * EMBED_DIM
PPW = PLANES // NW
OCHUNK = 4096
NOC = BATCH // OCHUNK


@functools.partial(
    pl.kernel,
    mesh=plsc.VectorSubcoreMesh(core_axis_name="c", subcore_axis_name="s"),
    out_type=jax.ShapeDtypeStruct((NUM_FIELDS, EMBED_DIM, BATCH),
                                  jnp.float32),
    compiler_params=pltpu.CompilerParams(needs_layout_passes=False),
    scratch_types=[
        pltpu.VMEM((BATCH,), jnp.int32),
        pltpu.VMEM((VOCAB,), jnp.float32),
        pltpu.VMEM((PPW * L,), jnp.float32),
        pltpu.VMEM((OCHUNK,), jnp.float32),
        pltpu.VMEM((OCHUNK,), jnp.float32),
        pltpu.SemaphoreType.DMA,
        pltpu.SemaphoreType.DMA,
    ],
)
def _embed_sc(idx_hbm, table_hbm, bias_hbm, out_hbm,
              idx_v, plane_v, bias_v, out_a, out_b, sem_a, sem_b):
    wid = lax.axis_index("s") * NC + lax.axis_index("c")
    p0 = wid * PPW
    pltpu.sync_copy(bias_hbm.at[pl.ds(p0 * L, PPW * L)], bias_v)

    def plane_body(k, f_prev):
        p = p0 + k
        f = p // EMBED_DIM
        d = lax.rem(p, EMBED_DIM)

        @pl.when(f != f_prev)
        def _():
            pltpu.sync_copy(idx_hbm.at[f, :], idx_v)

        pltpu.sync_copy(table_hbm.at[f, d, :], plane_v)
        bvec = bias_v[pl.ds(k * L, L)]

        stores = []
        for c in range(NOC):
            base = c * OCHUNK
            out_v = out_a if c % 2 == 0 else out_b
            sem = sem_a if c % 2 == 0 else sem_b
            if c >= 2:
                stores[c - 2].wait()

            @plsc.parallel_loop(0, OCHUNK, step=L, unroll=8)
            def _(i):
                iv = idx_v[pl.ds(base + i, L)]
                vals = plsc.load_gather(plane_v, [iv])
                out_v[pl.ds(i, L)] = vals + bvec

            stores.append(pltpu.async_copy(
                out_v, out_hbm.at[f, d, pl.ds(base, OCHUNK)], sem))
        stores[-2].wait()
        stores[-1].wait()
        return f

    lax.fori_loop(0, PPW, plane_body, jnp.int32(-1))


def kernel(categorical_inputs, tables, column_embedding):
    table_t = jnp.transpose(tables, (0, 2, 1))
    idx_t = jnp.transpose(categorical_inputs.astype(jnp.int32),
                          (1, 0))
    bias_b = jnp.broadcast_to(
        column_embedding.reshape(PLANES, 1), (PLANES, L)).reshape(-1)
    out_t = _embed_sc(idx_t, table_t, bias_b)
    return jnp.transpose(out_t, (2, 0, 1))
